# baseline (speedup 1.0000x reference)
import functools
import jax
import jax.numpy as jnp
from jax import lax
from jax.experimental import pallas as pl
from jax.experimental.pallas import tpu as pltpu
from jax.experimental.pallas import tpu_sc as plsc

EMB = 64
SCALE = 8.0
NC, NS = 2, 16
NW = NC * NS
CHUNK = 128
L = 16
NBUF = 4


def _make_kernel(T, S):
    n_tb = T // CHUNK
    n_st = S * n_tb
    spw = n_st // NW
    assert spw % NBUF == 0
    mesh = plsc.VectorSubcoreMesh(core_axis_name="c", subcore_axis_name="s")

    @functools.partial(
        pl.kernel,
        out_type=jax.ShapeDtypeStruct((S * (EMB // 8) * n_tb, 8 * CHUNK), jnp.float32),
        mesh=mesh,
        compiler_params=pltpu.CompilerParams(
            use_tc_tiling_on_sc=False, needs_layout_passes=False
        ),
        scratch_types=[
            pltpu.VMEM((spw, CHUNK), jnp.int32),
            pltpu.VMEM((NBUF, CHUNK, EMB), jnp.float32),
            pltpu.VMEM((NBUF, 8, 8 * CHUNK), jnp.float32),
            pltpu.SemaphoreType.DMA((NBUF,)),
            pltpu.SemaphoreType.DMA((NBUF,)),
        ],
    )
    def k(tok_hbm, tbl_hbm, out_hbm, tok_v, gbufs, obufs, gsem, wsem):
        wid = lax.axis_index("s") * NC + lax.axis_index("c")
        st0 = wid * spw
        pltpu.sync_copy(tok_hbm.at[pl.ds(st0, spw)], tok_v)

        iot = lax.iota(jnp.int32, L)

        for b in range(NBUF):
            pltpu.async_copy(
                tbl_hbm.at[tok_v.at[b]],
                gbufs.at[b],
                gsem.at[b],
            )

        @pl.loop(0, spw, step=NBUF)
        def outer(kblk):
            for b in range(NBUF):
                kk = kblk + b
                st = st0 + kk
                s = st // n_tb
                tb = lax.rem(st, n_tb)
                pltpu.make_async_copy(
                    tbl_hbm.at[tok_v.at[0]], gbufs.at[b], gsem.at[b]
                ).wait()

                @pl.when(kblk > 0)
                def _():
                    for eb in range(8):
                        pltpu.make_async_copy(
                            obufs.at[b, 0], out_hbm.at[0], wsem.at[b]
                        ).wait()

                @pl.loop(0, CHUNK // L)
                def rows(g):
                    rowi = g * L + iot
                    for e in range(EMB):
                        val = plsc.load_gather(gbufs.at[b], [rowi, iot * 0 + e])
                        obufs[b, e // 8, pl.ds((e % 8) * CHUNK + g * L, L)] = val * SCALE

                srow = (s * 8) * n_tb + tb
                for eb in range(8):
                    pltpu.async_copy(
                        obufs.at[b, eb], out_hbm.at[srow + eb * n_tb], wsem.at[b]
                    )

                kn = kk + NBUF

                @pl.when(kn < spw)
                def _():
                    pltpu.async_copy(
                        tbl_hbm.at[tok_v.at[kn]],
                        gbufs.at[b],
                        gsem.at[b],
                    )

        for b in range(NBUF):
            for eb in range(8):
                pltpu.make_async_copy(
                    obufs.at[b, 0], out_hbm.at[0], wsem.at[b]
                ).wait()

    return k


def kernel(tokens, table):
    T, S = tokens.shape
    tokT = jnp.transpose(tokens).reshape(-1, CHUNK).astype(jnp.int32)
    X = _make_kernel(T, S)(tokT, table)
    X = X.reshape(S, EMB // 8, T // CHUNK, 8, CHUNK)
    return X.transpose(2, 4, 0, 1, 3).reshape(T, S, EMB)


# speedup vs baseline: 2.5988x; 2.5988x over previous
import functools
import jax
import jax.numpy as jnp
from jax import lax
from jax.experimental import pallas as pl
from jax.experimental.pallas import tpu as pltpu
from jax.experimental.pallas import tpu_sc as plsc

EMB = 64
SCALE = 8.0
NC, NS = 2, 16
NW = NC * NS
CHUNK = 128
L = 16
NBUF = 4


def _make_kernel(T, S):
    n_tb = T // CHUNK
    n_st = S * n_tb
    spw = n_st // NW
    assert spw % NBUF == 0
    mesh = plsc.VectorSubcoreMesh(core_axis_name="c", subcore_axis_name="s")

    @functools.partial(
        pl.kernel,
        out_type=jax.ShapeDtypeStruct((S * (EMB // 8) * n_tb, 8 * CHUNK), jnp.float32),
        mesh=mesh,
        compiler_params=pltpu.CompilerParams(
            use_tc_tiling_on_sc=False, needs_layout_passes=False
        ),
        scratch_types=[
            pltpu.VMEM((spw, CHUNK), jnp.int32),
            pltpu.VMEM((NBUF, CHUNK, EMB), jnp.float32),
            pltpu.VMEM((NBUF, 8 * 8 * CHUNK), jnp.float32),
            pltpu.SemaphoreType.DMA((NBUF,)),
            pltpu.SemaphoreType.DMA((NBUF,)),
        ],
    )
    def k(tok_hbm, tbl_hbm, out_hbm, tok_v, gbufs, obufs, gsem, wsem):
        wid = lax.axis_index("s") * NC + lax.axis_index("c")
        st0 = wid * spw
        pltpu.sync_copy(tok_hbm.at[pl.ds(st0, spw)], tok_v)

        iot = lax.iota(jnp.int32, L)
        rowv = [g * L + iot for g in range(CHUNK // L)]

        for b in range(NBUF):
            pltpu.async_copy(
                tbl_hbm.at[tok_v.at[b]],
                gbufs.at[b],
                gsem.at[b],
            )

        @pl.loop(0, spw, step=NBUF)
        def outer(kblk):
            for b in range(NBUF):
                kk = kblk + b
                st = st0 + kk
                s = st // n_tb
                tb = lax.rem(st, n_tb)
                pltpu.make_async_copy(
                    tbl_hbm.at[tok_v.at[0]], gbufs.at[b], gsem.at[b]
                ).wait()

                @pl.when(kblk > 0)
                def _():
                    for eb in range(8):
                        pltpu.make_async_copy(
                            obufs.at[b, pl.ds(0, 1024)], out_hbm.at[0], wsem.at[b]
                        ).wait()

                @functools.partial(plsc.parallel_loop, 0, EMB, unroll=2)
                def etrans(e):
                    base = (e // 8) * (8 * CHUNK) + lax.rem(e, 8) * CHUNK
                    cole = iot * 0 + e
                    for g in range(CHUNK // L):
                        val = plsc.load_gather(gbufs.at[b], [rowv[g], cole])
                        obufs[b, pl.ds(base + g * L, L)] = val * SCALE

                srow = (s * 8) * n_tb + tb
                for eb in range(8):
                    pltpu.async_copy(
                        obufs.at[b, pl.ds(eb * 1024, 1024)],
                        out_hbm.at[srow + eb * n_tb],
                        wsem.at[b],
                    )

                kn = kk + NBUF

                @pl.when(kn < spw)
                def _():
                    pltpu.async_copy(
                        tbl_hbm.at[tok_v.at[kn]],
                        gbufs.at[b],
                        gsem.at[b],
                    )

        for b in range(NBUF):
            for eb in range(8):
                pltpu.make_async_copy(
                    obufs.at[b, pl.ds(0, 1024)], out_hbm.at[0], wsem.at[b]
                ).wait()

    return k


def kernel(tokens, table):
    T, S = tokens.shape
    tokT = jnp.transpose(tokens).reshape(-1, CHUNK).astype(jnp.int32)
    X = _make_kernel(T, S)(tokT, table)
    X = X.reshape(S, EMB // 8, T // CHUNK, 8, CHUNK)
    return X.transpose(2, 4, 0, 1, 3).reshape(T, S, EMB)
